# baseline (speedup 1.0000x reference)
import functools

import jax
import jax.numpy as jnp
from jax.experimental import pallas as pl
from jax.experimental.pallas import tpu as pltpu

B, S, D, A, E = 4, 2048, 512, 64, 4
N = B * S

XBLK = 1024
RBLK = 8


def _k1_body(x_ref, wf_ref, bf_ref, wg_ref, bg_ref, adapt_ref, gate_ref,
             acc_ref):
    i = pl.program_id(0)

    @pl.when(i == 0)
    def _():
        acc_ref[...] = jnp.zeros_like(acc_ref)

    acc_ref[...] += jnp.sum(x_ref[...], axis=0, keepdims=True)

    @pl.when(i == pl.num_programs(0) - 1)
    def _():
        mean = acc_ref[...] * (1.0 / N)
        adapt = jax.lax.dot_general(
            mean, wf_ref[...], (((1,), (1,)), ((), ())),
            preferred_element_type=jnp.float32,
            precision=jax.lax.Precision.HIGHEST) + bf_ref[...]
        logits = jax.lax.dot_general(
            adapt, wg_ref[...], (((1,), (1,)), ((), ())),
            preferred_element_type=jnp.float32,
            precision=jax.lax.Precision.HIGHEST) + bg_ref[...]
        m = jnp.max(logits, axis=-1, keepdims=True)
        eg = jnp.exp(logits - m)
        gate_ref[...] = eg / jnp.sum(eg, axis=-1, keepdims=True)
        adapt_ref[...] = adapt


def _k2_body(adapt_ref, gate_ref,
             tmW_ref, tmAW_ref, tmAb_ref,
             cmW_ref, cmAW_ref, cmAb_ref,
             exW_ref, exAW_ref, exAb_ref, exb_ref,
             tmb_ref, cmb_ref, wout_ref, bout_ref,
             wmem_ref, bmem_ref, wtime_ref, btime_ref,
             v_ref, c_ref,
             tmw_s, cmw_s, comb_s):
    i = pl.program_id(0)
    r0 = i * RBLK
    a3 = adapt_ref[...].reshape(1, 1, A)

    tmw_s[pl.ds(r0, RBLK), :] = (
        tmW_ref[...] + tmAb_ref[...]
        + jnp.sum(tmAW_ref[...] * a3, axis=2))
    cmw_s[pl.ds(r0, RBLK), :] = (
        cmW_ref[...] + cmAb_ref[...]
        + jnp.sum(cmAW_ref[...] * a3, axis=2))

    ew = (exW_ref[...] + exAb_ref[...]
          + jnp.sum(exAW_ref[...] * a3.reshape(1, 1, 1, A), axis=3))
    acc = gate_ref[0, 0] * ew[0]
    for e in range(1, E):
        acc += gate_ref[0, e] * ew[e]
    comb_s[pl.ds(r0, RBLK), :] = acc

    @pl.when(i == pl.num_programs(0) - 1)
    def _():
        gate = gate_ref[...]
        bcomb = jax.lax.dot_general(
            gate, exb_ref[...], (((1,), (0,)), ((), ())),
            preferred_element_type=jnp.float32,
            precision=jax.lax.Precision.HIGHEST)

        def chain(u0, c0):
            u1 = jnp.dot(u0, wout_ref[...],
                         preferred_element_type=jnp.float32,
            precision=jax.lax.Precision.HIGHEST)
            c1 = c0 + jnp.sum(u0 * bout_ref[...], keepdims=True)[:, :1]
            u2 = jnp.dot(u1, comb_s[...],
                         preferred_element_type=jnp.float32,
            precision=jax.lax.Precision.HIGHEST)
            c2 = c1 + jnp.sum(u1 * bcomb, keepdims=True)[:, :1]
            u3 = jnp.dot(u2, cmw_s[...],
                         preferred_element_type=jnp.float32,
            precision=jax.lax.Precision.HIGHEST)
            c3 = c2 + jnp.sum(u2 * cmb_ref[...], keepdims=True)[:, :1]
            u4 = jnp.dot(u3, tmw_s[...],
                         preferred_element_type=jnp.float32,
            precision=jax.lax.Precision.HIGHEST)
            c4 = c3 + jnp.sum(u3 * tmb_ref[...], keepdims=True)[:, :1]
            return u4, c4

        vm, cm = chain(wmem_ref[...], bmem_ref[...])
        vt, ct = chain(wtime_ref[...], btime_ref[...])
        v_ref[0:1, :] = vm
        v_ref[1:2, :] = vt
        c_ref[:, 0:1] = cm
        c_ref[:, 1:2] = ct


def _k3_body(x_ref, v_ref, c_ref, out_ref):
    out_ref[...] = jax.lax.dot_general(
        x_ref[...], v_ref[...], (((1,), (1,)), ((), ())),
        preferred_element_type=jnp.float32,
            precision=jax.lax.Precision.HIGHEST) + c_ref[...]


@jax.jit
def kernel(x, W_feat, b_feat, tm_W, tm_b, tm_AW, tm_Ab, cm_W, cm_b, cm_AW,
           cm_Ab, ex_W, ex_b, ex_AW, ex_Ab, W_gate, b_gate, W_out, b_out,
           W_mem, b_mem, W_time, b_time):
    xf = x.reshape(N, D)

    adapt, gate = pl.pallas_call(
        _k1_body,
        grid=(N // XBLK,),
        in_specs=[
            pl.BlockSpec((XBLK, D), lambda i: (i, 0)),
            pl.BlockSpec((A, D), lambda i: (0, 0)),
            pl.BlockSpec((1, A), lambda i: (0, 0)),
            pl.BlockSpec((E, A), lambda i: (0, 0)),
            pl.BlockSpec((1, E), lambda i: (0, 0)),
        ],
        out_specs=[
            pl.BlockSpec((1, A), lambda i: (0, 0)),
            pl.BlockSpec((1, E), lambda i: (0, 0)),
        ],
        out_shape=[
            jax.ShapeDtypeStruct((1, A), jnp.float32),
            jax.ShapeDtypeStruct((1, E), jnp.float32),
        ],
        scratch_shapes=[pltpu.VMEM((1, D), jnp.float32)],
    )(xf, W_feat, b_feat.reshape(1, A), W_gate, b_gate.reshape(1, E))

    vrow = lambda i: (0, 0)
    v, c = pl.pallas_call(
        _k2_body,
        grid=(D // RBLK,),
        in_specs=[
            pl.BlockSpec((1, A), vrow),
            pl.BlockSpec((1, E), vrow),
            pl.BlockSpec((RBLK, D), lambda i: (i, 0)),
            pl.BlockSpec((RBLK, D, A), lambda i: (i, 0, 0)),
            pl.BlockSpec((RBLK, D), lambda i: (i, 0)),
            pl.BlockSpec((RBLK, D), lambda i: (i, 0)),
            pl.BlockSpec((RBLK, D, A), lambda i: (i, 0, 0)),
            pl.BlockSpec((RBLK, D), lambda i: (i, 0)),
            pl.BlockSpec((E, RBLK, D), lambda i: (0, i, 0)),
            pl.BlockSpec((E, RBLK, D, A), lambda i: (0, i, 0, 0)),
            pl.BlockSpec((E, RBLK, D), lambda i: (0, i, 0)),
            pl.BlockSpec((E, D), vrow),
            pl.BlockSpec((1, D), vrow),
            pl.BlockSpec((1, D), vrow),
            pl.BlockSpec((D, D), vrow),
            pl.BlockSpec((1, D), vrow),
            pl.BlockSpec((1, D), vrow),
            pl.BlockSpec((1, 1), vrow),
            pl.BlockSpec((1, D), vrow),
            pl.BlockSpec((1, 1), vrow),
        ],
        out_specs=[
            pl.BlockSpec((2, D), vrow),
            pl.BlockSpec((1, 2), vrow),
        ],
        out_shape=[
            jax.ShapeDtypeStruct((2, D), jnp.float32),
            jax.ShapeDtypeStruct((1, 2), jnp.float32),
        ],
        scratch_shapes=[
            pltpu.VMEM((D, D), jnp.float32),
            pltpu.VMEM((D, D), jnp.float32),
            pltpu.VMEM((D, D), jnp.float32),
        ],
    )(adapt, gate,
      tm_W, tm_AW.reshape(D, D, A), tm_Ab.reshape(D, D),
      cm_W, cm_AW.reshape(D, D, A), cm_Ab.reshape(D, D),
      ex_W, ex_AW.reshape(E, D, D, A), ex_Ab.reshape(E, D, D), ex_b,
      tm_b.reshape(1, D), cm_b.reshape(1, D), W_out, b_out.reshape(1, D),
      W_mem, b_mem.reshape(1, 1), W_time, b_time.reshape(1, 1))

    out = pl.pallas_call(
        _k3_body,
        grid=(N // XBLK,),
        in_specs=[
            pl.BlockSpec((XBLK, D), lambda i: (i, 0)),
            pl.BlockSpec((2, D), lambda i: (0, 0)),
            pl.BlockSpec((1, 2), lambda i: (0, 0)),
        ],
        out_specs=pl.BlockSpec((XBLK, 2), lambda i: (i, 0)),
        out_shape=jax.ShapeDtypeStruct((N, 2), jnp.float32),
    )(xf, v, c)

    mem_pred = out[:, 0].reshape(B, S)
    time_pred = out[:, 1].reshape(B, S)
    return (mem_pred, time_pred)
